# Initial kernel scaffold; baseline (speedup 1.0000x reference)
import jax
import jax.numpy as jnp
from jax.experimental import pallas as pl


def kernel(x, edge_index, batch, W1, b1, W2, b2, W3, b3, Wm1, bm1, Wm2, bm2, Wh1, bh1, Wh2, bh2):
    raise NotImplementedError("write your pallas kernel here")


# speedup vs baseline: 11.3941x; 11.3941x over previous
import functools

import jax
import jax.numpy as jnp
from jax import lax
from jax.experimental import pallas as pl
from jax.experimental.pallas import tpu as pltpu
from jax.experimental.pallas import tpu_sc as plsc

N = 10000
NP = 10240
E = 320000
G = 256
D = 128

_KC = 80
_NSUB = 16
_NCORE = 2
_EPT = E // (_NCORE * _NSUB)
_NCH = _EPT // _KC
_RPT = NP // _NSUB


@functools.lru_cache(maxsize=None)
def _sc_kernels():
    mesh = plsc.VectorSubcoreMesh(core_axis_name="c", subcore_axis_name="s")

    @functools.partial(
        pl.kernel,
        mesh=mesh,
        out_type=jax.ShapeDtypeStruct((_NCORE, NP), jnp.float32),
        scratch_types=[
            pltpu.VMEM((_KC,), jnp.int32),
            pltpu.VMEM((_KC,), jnp.float32),
            pltpu.VMEM_SHARED((NP,), jnp.float32),
        ],
    )
    def sc_degree(dst_hbm, zeros_hbm, out_hbm, idx_v, ones_v, acc):
        c = lax.axis_index("c")
        s = lax.axis_index("s")
        for i in range(_KC // 16):
            ones_v[pl.ds(i * 16, 16)] = jnp.ones((16,), jnp.float32)

        @pl.when(s == 0)
        def _zero():
            pltpu.sync_copy(zeros_hbm, acc)

        plsc.subcore_barrier()
        base = (c * _NSUB + s) * _EPT

        def body(j, carry):
            off = pl.multiple_of(base + j * _KC, 8)
            pltpu.sync_copy(dst_hbm.at[pl.ds(off, _KC)], idx_v)
            pltpu.sync_copy(ones_v, acc.at[idx_v], add=True)
            return carry

        lax.fori_loop(0, _NCH, body, 0)
        plsc.subcore_barrier()
        pltpu.sync_copy(acc.at[pl.ds(s * _RPT, _RPT)],
                        out_hbm.at[c, pl.ds(s * _RPT, _RPT)])

    @functools.partial(
        pl.kernel,
        mesh=mesh,
        out_type=jax.ShapeDtypeStruct((_NCORE, NP, D), jnp.float32),
        scratch_types=[
            pltpu.VMEM((_KC,), jnp.int32),
            pltpu.VMEM((_KC,), jnp.int32),
            pltpu.VMEM((_KC, D), jnp.float32),
            pltpu.VMEM_SHARED((NP, D), jnp.float32),
            pltpu.SemaphoreType.DMA,
        ],
    )
    def sc_propagate(table_hbm, src_hbm, dst_hbm, zeros_hbm, out_hbm,
                     si_v, di_v, rows_v, acc, sem):
        c = lax.axis_index("c")
        s = lax.axis_index("s")

        @pl.when(s == 0)
        def _zero():
            pltpu.sync_copy(zeros_hbm, acc)

        plsc.subcore_barrier()
        base = (c * _NSUB + s) * _EPT

        def body(j, carry):
            off = pl.multiple_of(base + j * _KC, 8)
            pltpu.sync_copy(src_hbm.at[pl.ds(off, _KC)], si_v)
            pltpu.sync_copy(dst_hbm.at[pl.ds(off, _KC)], di_v)
            pltpu.async_copy(table_hbm.at[si_v], rows_v, sem).wait()
            pltpu.sync_copy(rows_v, acc.at[di_v], add=True)
            return carry

        lax.fori_loop(0, _NCH, body, 0)
        plsc.subcore_barrier()
        pltpu.sync_copy(acc.at[pl.ds(s * _RPT, _RPT)],
                        out_hbm.at[c, pl.ds(s * _RPT, _RPT)])

    return sc_degree, sc_propagate


def _tc_layer1(x, W1, degp):
    def body(x_ref, w_ref, degp_ref, scaled_ref, dinv_ref):
        dp = degp_ref[...]
        deg = dp[0, :N] + dp[1, :N] + 1.0
        dinv = lax.rsqrt(deg)
        dinv_ref[...] = dinv
        hw = jnp.dot(x_ref[...], w_ref[...], preferred_element_type=jnp.float32)
        scaled_ref[...] = hw * dinv

    return pl.pallas_call(
        body,
        out_shape=(jax.ShapeDtypeStruct((N, D), jnp.float32),
                   jax.ShapeDtypeStruct((N, 1), jnp.float32)),
    )(x, W1, degp)


def _tc_mid(tp, scaled_prev, dinv, b_prev, W):
    def body(tp_ref, sc_ref, dinv_ref, b_ref, w_ref, out_ref):
        tp_ = tp_ref[...]
        t = tp_[0, :N] + tp_[1, :N]
        dinv_ = dinv_ref[...]
        h = jnp.maximum(dinv_ * (t + sc_ref[...]) + b_ref[...], 0.0)
        out_ref[...] = jnp.dot(h, w_ref[...],
                               preferred_element_type=jnp.float32) * dinv_

    return pl.pallas_call(
        body,
        out_shape=jax.ShapeDtypeStruct((N, D), jnp.float32),
    )(tp, scaled_prev, dinv, b_prev, W)


def _tc_final(tp, scaled_prev, dinv, b_prev, batch2d,
              Wm1, bm1, Wm2, bm2, Wh1, bh1, Wh2, bh2):
    def body(tp_ref, sc_ref, dinv_ref, b_ref, batch_ref,
             wm1_ref, bm1_ref, wm2_ref, bm2_ref,
             wh1_ref, bh1_ref, wh2_ref, bh2_ref, out_ref):
        tp_ = tp_ref[...]
        t = tp_[0, :N] + tp_[1, :N]
        h = dinv_ref[...] * (t + sc_ref[...]) + b_ref[...]
        gids = lax.broadcasted_iota(jnp.int32, (N, G), 1)
        onehot = (batch_ref[...] == gids).astype(jnp.float32)
        dn = (((0,), (0,)), ((), ()))
        sums = lax.dot_general(onehot, h, dn,
                               preferred_element_type=jnp.float32)
        counts = lax.dot_general(onehot, jnp.ones((N, 1), jnp.float32), dn,
                                 preferred_element_type=jnp.float32)
        pooled = sums / jnp.maximum(counts, 1.0)
        z = jnp.maximum(jnp.dot(pooled, wm1_ref[...],
                                preferred_element_type=jnp.float32)
                        + bm1_ref[...], 0.0)
        z = jnp.maximum(jnp.dot(z, wm2_ref[...],
                                preferred_element_type=jnp.float32)
                        + bm2_ref[...], 0.0)
        r = jnp.maximum(jnp.dot(z, wh1_ref[...],
                                preferred_element_type=jnp.float32)
                        + bh1_ref[...], 0.0)
        out_ref[...] = jnp.dot(r, wh2_ref[...],
                               preferred_element_type=jnp.float32) + bh2_ref[...]

    return pl.pallas_call(
        body,
        out_shape=jax.ShapeDtypeStruct((G, D), jnp.float32),
    )(tp, scaled_prev, dinv, b_prev, batch2d,
      Wm1, bm1, Wm2, bm2, Wh1, bh1, Wh2, bh2)


def kernel(x, edge_index, batch, W1, b1, W2, b2, W3, b3,
           Wm1, bm1, Wm2, bm2, Wh1, bh1, Wh2, bh2):
    sc_degree, sc_propagate = _sc_kernels()
    src = edge_index[0].astype(jnp.int32)
    dst = edge_index[1].astype(jnp.int32)
    z1 = jnp.zeros((NP,), jnp.float32)
    z2 = jnp.zeros((NP, D), jnp.float32)

    degp = sc_degree(dst, z1).reshape(_NCORE, NP, 1)
    scaled1, dinv = _tc_layer1(x, W1, degp)
    t1 = sc_propagate(scaled1, src, dst, z2)
    scaled2 = _tc_mid(t1, scaled1, dinv, b1.reshape(1, D), W2)
    t2 = sc_propagate(scaled2, src, dst, z2)
    scaled3 = _tc_mid(t2, scaled2, dinv, b2.reshape(1, D), W3)
    t3 = sc_propagate(scaled3, src, dst, z2)
    return _tc_final(t3, scaled3, dinv, b3.reshape(1, D),
                     batch.astype(jnp.int32).reshape(N, 1),
                     Wm1, bm1.reshape(1, -1), Wm2, bm2.reshape(1, -1),
                     Wh1, bh1.reshape(1, -1), Wh2, bh2.reshape(1, -1))
